# Initial kernel scaffold; baseline (speedup 1.0000x reference)
import functools

import jax
import jax.numpy as jnp
from jax.experimental import pallas as pl

E = 8
TOPK = 2
D = 2048
DFF = 4096
T = 2048

F_TILE = 512


def _routing_kernel(x_ref, wg_ref, w_ref):
    logits = jax.lax.dot(x_ref[...], wg_ref[...],
                         precision=jax.lax.Precision.HIGHEST,
                         preferred_element_type=jnp.float32)
    gates = jax.nn.softmax(logits, axis=1)
    iota = jax.lax.broadcasted_iota(jnp.int32, (T, E), 1)
    m1 = jnp.max(gates, axis=1, keepdims=True)
    a1 = jnp.min(jnp.where(gates == m1, iota, E), axis=1, keepdims=True)
    g2 = jnp.where(iota == a1, -jnp.inf, gates)
    m2 = jnp.max(g2, axis=1, keepdims=True)
    a2 = jnp.min(jnp.where(g2 == m2, iota, E), axis=1, keepdims=True)
    denom = jnp.maximum(m1 + m2, 1.1920929e-07)
    sel = (iota == a1) | (iota == a2)
    w_ref[...] = jnp.where(sel, gates / denom, 0.0)


def _moe_dense_kernel(xb_ref, w_ref, wg_ref, wu_ref, wd_ref, out_ref):
    e = pl.program_id(0)
    f = pl.program_id(1)

    xb = xb_ref[...]
    h = jax.lax.dot(xb, wg_ref[0], preferred_element_type=jnp.float32)
    u = jax.lax.dot(xb, wu_ref[0], preferred_element_type=jnp.float32)
    inter = jax.nn.silu(h) * u
    scale = w_ref[0]
    inter = (inter * scale).astype(jnp.bfloat16)
    contrib = jax.lax.dot(inter, wd_ref[0], preferred_element_type=jnp.float32)

    @pl.when((e == 0) & (f == 0))
    def _init():
        out_ref[...] = contrib

    @pl.when((e > 0) | (f > 0))
    def _acc():
        out_ref[...] += contrib


@functools.partial(jax.jit)
def kernel(x, wg, w_gate, w_up, w_down):
    w_sel = pl.pallas_call(
        _routing_kernel,
        out_shape=jax.ShapeDtypeStruct((T, E), jnp.float32),
    )(x, wg)

    w_sel_t = w_sel.T.reshape(E, T, 1)
    xb = x.astype(jnp.bfloat16)
    wgb = w_gate.astype(jnp.bfloat16)
    wub = w_up.astype(jnp.bfloat16)
    wdb = w_down.astype(jnp.bfloat16)

    nf = DFF // F_TILE
    out = pl.pallas_call(
        _moe_dense_kernel,
        grid=(E, nf),
        in_specs=[
            pl.BlockSpec((T, D), lambda e, f: (0, 0)),
            pl.BlockSpec((1, T, 1), lambda e, f: (e, 0, 0)),
            pl.BlockSpec((1, D, F_TILE), lambda e, f: (e, 0, f)),
            pl.BlockSpec((1, D, F_TILE), lambda e, f: (e, 0, f)),
            pl.BlockSpec((1, F_TILE, D), lambda e, f: (e, f, 0)),
        ],
        out_specs=pl.BlockSpec((T, D), lambda e, f: (0, 0)),
        out_shape=jax.ShapeDtypeStruct((T, D), jnp.float32),
    )(xb, w_sel_t, wgb, wub, wdb)
    return out


# speedup vs baseline: 1.0499x; 1.0499x over previous
import functools

import jax
import jax.numpy as jnp
from jax.experimental import pallas as pl

E = 8
TOPK = 2
D = 2048
DFF = 4096
T = 2048

F_TILE = 512


def _routing_kernel(x_ref, wg_ref, w_ref):
    logits = jax.lax.dot(x_ref[...], wg_ref[...],
                         preferred_element_type=jnp.float32)
    gates = jax.nn.softmax(logits, axis=1)
    iota = jax.lax.broadcasted_iota(jnp.int32, (T, E), 1)
    m1 = jnp.max(gates, axis=1, keepdims=True)
    a1 = jnp.min(jnp.where(gates == m1, iota, E), axis=1, keepdims=True)
    g2 = jnp.where(iota == a1, -jnp.inf, gates)
    m2 = jnp.max(g2, axis=1, keepdims=True)
    a2 = jnp.min(jnp.where(g2 == m2, iota, E), axis=1, keepdims=True)
    denom = jnp.maximum(m1 + m2, 1.1920929e-07)
    sel = (iota == a1) | (iota == a2)
    w_ref[...] = jnp.where(sel, gates / denom, 0.0)


def _moe_dense_kernel(xb_ref, w_ref, wg_ref, wu_ref, wd_ref, out_ref):
    e = pl.program_id(0)
    f = pl.program_id(1)

    xb = xb_ref[...]
    h = jax.lax.dot(xb, wg_ref[0], preferred_element_type=jnp.float32)
    u = jax.lax.dot(xb, wu_ref[0], preferred_element_type=jnp.float32)
    inter = jax.nn.silu(h) * u
    scale = w_ref[0]
    inter = (inter * scale).astype(jnp.bfloat16)
    contrib = jax.lax.dot(inter, wd_ref[0], preferred_element_type=jnp.float32)

    @pl.when((e == 0) & (f == 0))
    def _init():
        out_ref[...] = contrib

    @pl.when((e > 0) | (f > 0))
    def _acc():
        out_ref[...] += contrib


@functools.partial(jax.jit)
def kernel(x, wg, w_gate, w_up, w_down):
    w_sel = pl.pallas_call(
        _routing_kernel,
        out_shape=jax.ShapeDtypeStruct((T, E), jnp.float32),
    )(x, wg)

    w_sel_t = w_sel.T.reshape(E, T, 1)
    xb = x.astype(jnp.bfloat16)
    wgb = w_gate.astype(jnp.bfloat16)
    wub = w_up.astype(jnp.bfloat16)
    wdb = w_down.astype(jnp.bfloat16)

    nf = DFF // F_TILE
    out = pl.pallas_call(
        _moe_dense_kernel,
        grid=(E, nf),
        in_specs=[
            pl.BlockSpec((T, D), lambda e, f: (0, 0)),
            pl.BlockSpec((1, T, 1), lambda e, f: (e, 0, 0)),
            pl.BlockSpec((1, D, F_TILE), lambda e, f: (e, 0, f)),
            pl.BlockSpec((1, D, F_TILE), lambda e, f: (e, 0, f)),
            pl.BlockSpec((1, F_TILE, D), lambda e, f: (e, f, 0)),
        ],
        out_specs=pl.BlockSpec((T, D), lambda e, f: (0, 0)),
        out_shape=jax.ShapeDtypeStruct((T, D), jnp.float32),
    )(xb, w_sel_t, wgb, wub, wdb)
    return out


# speedup vs baseline: 1.3619x; 1.2972x over previous
import functools

import jax
import jax.numpy as jnp
from jax import lax
from jax.experimental import pallas as pl
from jax.experimental.pallas import tpu as pltpu
from jax.experimental.pallas import tpu_sc as plsc

E = 8
D = 2048
DFF = 4096
T = 2048

BLK = 256
G = (2 * T) // BLK + E
GB = G * BLK
F_TILE = 1024
NF = DFF // F_TILE

NW = 32
ROWS_PER_W = GB // NW
CH = 32
T_PER_W = T // NW
CH2 = 16


def _routing_kernel(x_ref, wg_ref, ids_ref, probs_ref):
    logits = jax.lax.dot(x_ref[...], wg_ref[...],
                         preferred_element_type=jnp.float32)
    gates = jax.nn.softmax(logits, axis=1)
    iota = jax.lax.broadcasted_iota(jnp.int32, (T, E), 1)
    m1 = jnp.max(gates, axis=1, keepdims=True)
    a1 = jnp.min(jnp.where(gates == m1, iota, E), axis=1, keepdims=True)
    g2 = jnp.where(iota == a1, -jnp.inf, gates)
    m2 = jnp.max(g2, axis=1, keepdims=True)
    a2 = jnp.min(jnp.where(g2 == m2, iota, E), axis=1, keepdims=True)
    denom = jnp.maximum(m1 + m2, 1.1920929e-07)
    ids_ref[...] = jnp.concatenate([a1, a2], axis=1)
    probs_ref[...] = jnp.concatenate([m1 / denom, m2 / denom], axis=1)


def _routing(x, wg):
    return pl.pallas_call(
        _routing_kernel,
        out_shape=(jax.ShapeDtypeStruct((T, 2), jnp.int32),
                   jax.ShapeDtypeStruct((T, 2), jnp.float32)),
    )(x, wg)


def _build_indices(ids, probs):
    ef = ids.T.reshape(-1)
    pf = probs.T.reshape(-1)
    em = jax.nn.one_hot(ef, E, dtype=jnp.int32)
    csum = jnp.cumsum(em, axis=0)
    rank = jnp.sum(csum * em, axis=1) - 1
    counts = csum[-1]
    padded = ((counts + BLK - 1) // BLK) * BLK
    off = jnp.concatenate([jnp.zeros((1,), jnp.int32),
                           jnp.cumsum(padded)[:-1].astype(jnp.int32)])
    slot = off[ef] + rank
    tokens = jnp.concatenate([jnp.arange(T, dtype=jnp.int32)] * 2)
    row_token = jnp.zeros((GB,), jnp.int32).at[slot].set(tokens)
    row_w = jnp.zeros((GB,), jnp.float32).at[slot].set(pf)
    pend = jnp.cumsum(padded // BLK)
    block_expert = jnp.minimum(
        jnp.searchsorted(pend, jnp.arange(G), side='right'), E - 1
    ).astype(jnp.int32)
    pos0, pos1 = slot[:T], slot[T:]
    return row_token, row_w, block_expert, pos0.astype(jnp.int32), pos1.astype(jnp.int32)


def _dispatch(x, row_token):
    mesh = plsc.VectorSubcoreMesh(core_axis_name="c", subcore_axis_name="s")

    @functools.partial(
        pl.kernel, mesh=mesh,
        out_type=jax.ShapeDtypeStruct((GB, D), jnp.float32),
        scratch_types=[
            pltpu.VMEM((CH,), jnp.int32),
            pltpu.VMEM((CH, D), jnp.float32),
            pltpu.SemaphoreType.DMA,
        ],
    )
    def k(x_hbm, idx_hbm, out_hbm, idx_v, rows_v, sem):
        wid = lax.axis_index("s") * 2 + lax.axis_index("c")

        @pl.loop(0, ROWS_PER_W // CH)
        def _(c):
            base = wid * ROWS_PER_W + c * CH
            pltpu.sync_copy(idx_hbm.at[pl.ds(base, CH)], idx_v)
            pltpu.async_copy(x_hbm.at[idx_v], rows_v, sem).wait()
            pltpu.sync_copy(rows_v, out_hbm.at[pl.ds(base, CH)])

    return k(x, row_token)


def _ffn_kernel(be_ref, xs_ref, rw_ref, wg_ref, wu_ref, wd_ref, out_ref):
    f = pl.program_id(1)
    xb = xs_ref[...].astype(jnp.bfloat16)
    h = jax.lax.dot(xb, wg_ref[0], preferred_element_type=jnp.float32)
    u = jax.lax.dot(xb, wu_ref[0], preferred_element_type=jnp.float32)
    inter = (jax.nn.silu(h) * u * rw_ref[...]).astype(jnp.bfloat16)
    contrib = jax.lax.dot(inter, wd_ref[0], preferred_element_type=jnp.float32)

    @pl.when(f == 0)
    def _init():
        out_ref[...] = contrib

    @pl.when(f > 0)
    def _acc():
        out_ref[...] += contrib


def _ffn(block_expert, xs, row_w, wgb, wub, wdb):
    grid_spec = pltpu.PrefetchScalarGridSpec(
        num_scalar_prefetch=1,
        grid=(G, NF),
        in_specs=[
            pl.BlockSpec((BLK, D), lambda g, f, be: (g, 0)),
            pl.BlockSpec((BLK, 1), lambda g, f, be: (g, 0)),
            pl.BlockSpec((1, D, F_TILE), lambda g, f, be: (be[g], 0, f)),
            pl.BlockSpec((1, D, F_TILE), lambda g, f, be: (be[g], 0, f)),
            pl.BlockSpec((1, F_TILE, D), lambda g, f, be: (be[g], f, 0)),
        ],
        out_specs=pl.BlockSpec((BLK, D), lambda g, f, be: (g, 0)),
    )
    return pl.pallas_call(
        _ffn_kernel,
        grid_spec=grid_spec,
        out_shape=jax.ShapeDtypeStruct((GB, D), jnp.float32),
    )(block_expert, xs, row_w.reshape(GB, 1), wgb, wub, wdb)


def _combine(ys, pos0, pos1):
    mesh = plsc.VectorSubcoreMesh(core_axis_name="c", subcore_axis_name="s")

    @functools.partial(
        pl.kernel, mesh=mesh,
        out_type=jax.ShapeDtypeStruct((T, D), jnp.float32),
        scratch_types=[
            pltpu.VMEM((CH2,), jnp.int32),
            pltpu.VMEM((CH2,), jnp.int32),
            pltpu.VMEM((CH2, D), jnp.float32),
            pltpu.VMEM((CH2, D), jnp.float32),
            pltpu.SemaphoreType.DMA,
            pltpu.SemaphoreType.DMA,
        ],
    )
    def k(ys_hbm, p0_hbm, p1_hbm, out_hbm, i0_v, i1_v, r0_v, r1_v, s0, s1):
        wid = lax.axis_index("s") * 2 + lax.axis_index("c")

        @pl.loop(0, T_PER_W // CH2)
        def _(c):
            base = wid * T_PER_W + c * CH2
            pltpu.sync_copy(p0_hbm.at[pl.ds(base, CH2)], i0_v)
            pltpu.sync_copy(p1_hbm.at[pl.ds(base, CH2)], i1_v)
            cp0 = pltpu.async_copy(ys_hbm.at[i0_v], r0_v, s0)
            cp1 = pltpu.async_copy(ys_hbm.at[i1_v], r1_v, s1)
            cp0.wait()
            cp1.wait()

            @pl.loop(0, CH2)
            def _(r):
                @pl.loop(0, D // 16)
                def _(j):
                    sl = pl.ds(j * 16, 16)
                    r0_v.at[r, sl][...] = r0_v.at[r, sl][...] + r1_v.at[r, sl][...]

            pltpu.sync_copy(r0_v, out_hbm.at[pl.ds(base, CH2)])

    return k(ys, pos0, pos1)


@jax.jit
def kernel(x, wg, w_gate, w_up, w_down):
    ids, probs = _routing(x, wg)
    row_token, row_w, block_expert, pos0, pos1 = _build_indices(ids, probs)
    xs = _dispatch(x, row_token)
    wgb = w_gate.astype(jnp.bfloat16)
    wub = w_up.astype(jnp.bfloat16)
    wdb = w_down.astype(jnp.bfloat16)
    ys = _ffn(block_expert, xs, row_w, wgb, wub, wdb)
    return _combine(ys, pos0, pos1)
